# baseline (speedup 1.0000x reference)
import jax
import jax.numpy as jnp
from jax import lax
from jax.experimental import pallas as pl
from jax.experimental.pallas import tpu as pltpu
from jax.experimental.pallas import tpu_sc as plsc

EMB = 128
NG = EMB // 16
CHUNK = 128
ROWS_PER_WORKER = 2048
NCHUNK = ROWS_PER_WORKER // CHUNK
NBUF = 4
UNROLL = 4


def _hrm_sc(item_hbm, user_hbm, idx_hbm, uidx_hbm, iidx_hbm, out_hbm,
            idx_v, buf0, buf1, buf2, buf3, accv, shared, merged, uiv, iiv,
            urows, irows, outv, sem0, sem1, sem2, sem3, semu, semi):
    c = lax.axis_index("c")
    s = lax.axis_index("s")
    r = c * 16 + s

    pltpu.sync_copy(idx_hbm.at[r], idx_v)

    bufs = (buf0, buf1, buf2, buf3)
    sems = (sem0, sem1, sem2, sem3)

    for t in range(NBUF):
        pltpu.async_copy(item_hbm.at[idx_v.at[t]], bufs[t], sems[t])

    neg = jnp.full((16,), -jnp.inf, dtype=jnp.float32)
    acc0 = (neg,) * NG

    def outer(jj, acc):
        for t in range(NBUF):
            ci = jj * NBUF + t
            pltpu.make_async_copy(item_hbm.at[idx_v.at[ci]], bufs[t],
                                  sems[t]).wait()
            bt = bufs[t]

            def row_body(rr, a, bt=bt):
                base = rr * UNROLL
                for u in range(UNROLL):
                    a = tuple(
                        jnp.maximum(a[g], bt[base + u, pl.ds(16 * g, 16)])
                        for g in range(NG))
                return a

            acc = lax.fori_loop(0, CHUNK // UNROLL, row_body, acc)

            nxt = ci + NBUF

            @pl.when(nxt < NCHUNK)
            def _(t=t, nxt=nxt):
                pltpu.async_copy(item_hbm.at[idx_v.at[nxt]], bufs[t], sems[t])
        return acc

    acc = lax.fori_loop(0, NCHUNK // NBUF, outer, acc0)

    for g in range(NG):
        accv[pl.ds(16 * g, 16)] = acc[g]
    pltpu.sync_copy(accv, shared.at[s])
    plsc.subcore_barrier()

    @pl.when(s < 8)
    def _():
        pltpu.sync_copy(shared.at[pl.ds(2 * s, 2)], merged)
        pltpu.sync_copy(uidx_hbm.at[pl.ds(c * 8, 8)], uiv)
        pltpu.sync_copy(iidx_hbm.at[pl.ds(c * 8, 8)], iiv)
        cp_u = pltpu.async_copy(user_hbm.at[uiv.at[s]], urows, semu)

        cp_i = pltpu.async_copy(item_hbm.at[iiv.at[s]], irows, semi)
        cp_u.wait()
        cp_i.wait()

        lanes = lax.iota(jnp.int32, 16)
        tot = jnp.zeros((16,), jnp.float32)
        for g in range(NG):
            sl = pl.ds(16 * g, 16)
            pooled = jnp.maximum(merged[0, sl], merged[1, sl])
            hyb = jnp.maximum(pooled, urows[0, sl])
            tot = tot + hyb * irows[0, sl]
        for sh in (8, 4, 2, 1):
            tot = tot + tot.at[lanes ^ sh].get(mode="promise_in_bounds")
        outv[pl.ds(0, 16)] = tot
        pltpu.sync_copy(outv, out_hbm.at[c * 8 + s])


@jax.jit
def kernel(user_table, item_table, user_inputs, L_inputs, S_inputs,
           item_inputs):
    batch = user_inputs.shape[0]
    idx_all = jnp.concatenate(
        [L_inputs.astype(jnp.int32)[:, None, :],
         S_inputs.astype(jnp.int32)[:, None, :]], axis=1)
    idx_all = idx_all.reshape(2 * batch, NCHUNK, CHUNK)

    mesh = plsc.VectorSubcoreMesh(core_axis_name="c", subcore_axis_name="s")
    out = pl.kernel(
        _hrm_sc,
        out_type=jax.ShapeDtypeStruct((batch, EMB), jnp.float32),
        mesh=mesh,
        scratch_types=[
            pltpu.VMEM((NCHUNK, CHUNK), jnp.int32),
            pltpu.VMEM((CHUNK, EMB), jnp.float32),
            pltpu.VMEM((CHUNK, EMB), jnp.float32),
            pltpu.VMEM((CHUNK, EMB), jnp.float32),
            pltpu.VMEM((CHUNK, EMB), jnp.float32),
            pltpu.VMEM((EMB,), jnp.float32),
            pltpu.VMEM_SHARED((16, EMB), jnp.float32),
            pltpu.VMEM((2, EMB), jnp.float32),
            pltpu.VMEM((8, 1), jnp.int32),
            pltpu.VMEM((8, 1), jnp.int32),
            pltpu.VMEM((1, EMB), jnp.float32),
            pltpu.VMEM((1, EMB), jnp.float32),
            pltpu.VMEM((EMB,), jnp.float32),
            pltpu.SemaphoreType.DMA,
            pltpu.SemaphoreType.DMA,
            pltpu.SemaphoreType.DMA,
            pltpu.SemaphoreType.DMA,
            pltpu.SemaphoreType.DMA,
            pltpu.SemaphoreType.DMA,
        ],
    )(item_table, user_table, idx_all,
      user_inputs.astype(jnp.int32).reshape(batch, 1),
      item_inputs.astype(jnp.int32).reshape(batch, 1))
    return out[:, :1]


# speedup vs baseline: 1.1109x; 1.1109x over previous
import jax
import jax.numpy as jnp
from jax import lax
from jax.experimental import pallas as pl
from jax.experimental.pallas import tpu as pltpu
from jax.experimental.pallas import tpu_sc as plsc

EMB = 128
NG = EMB // 16
CHUNK = 128
ROWS_PER_WORKER = 2048
NCHUNK = ROWS_PER_WORKER // CHUNK
NBUF = 4
UNROLL = 4


def _hrm_sc(item_hbm, user_hbm, idx_hbm, uidx_hbm, iidx_hbm, out_hbm,
            idx_v, buf0, buf1, buf2, buf3, accv, shared, merged, uiv, iiv,
            urows, irows, outv, sem0, sem1, sem2, sem3, semu, semi):
    c = lax.axis_index("c")
    s = lax.axis_index("s")
    r = c * 16 + s

    pltpu.sync_copy(idx_hbm.at[r], idx_v)

    bufs = (buf0, buf1, buf2, buf3)
    sems = (sem0, sem1, sem2, sem3)

    for t in range(NBUF):
        pltpu.async_copy(item_hbm.at[idx_v.at[t]], bufs[t], sems[t])

    @pl.when(s == 0)
    def _():
        pltpu.sync_copy(uidx_hbm.at[pl.ds(c * 8, 8)], uiv)
        pltpu.sync_copy(iidx_hbm.at[pl.ds(c * 8, 8)], iiv)
        pltpu.async_copy(user_hbm.at[uiv], urows, semu)
        pltpu.async_copy(item_hbm.at[iiv], irows, semi)

    neg = jnp.full((16,), -jnp.inf, dtype=jnp.float32)
    acc0 = (neg,) * NG

    def outer(jj, acc):
        for t in range(NBUF):
            ci = jj * NBUF + t
            pltpu.make_async_copy(item_hbm.at[idx_v.at[ci]], bufs[t],
                                  sems[t]).wait()
            bt = bufs[t]

            def row_body(rr, a, bt=bt):
                base = rr * UNROLL
                for u in range(UNROLL):
                    a = tuple(
                        jnp.maximum(a[g], bt[base + u, pl.ds(16 * g, 16)])
                        for g in range(NG))
                return a

            acc = lax.fori_loop(0, CHUNK // UNROLL, row_body, acc)

            nxt = ci + NBUF

            @pl.when(nxt < NCHUNK)
            def _(t=t, nxt=nxt):
                pltpu.async_copy(item_hbm.at[idx_v.at[nxt]], bufs[t], sems[t])
        return acc

    acc = lax.fori_loop(0, NCHUNK // NBUF, outer, acc0)

    for g in range(NG):
        accv[pl.ds(16 * g, 16)] = acc[g]
    pltpu.sync_copy(accv, shared.at[s])
    plsc.subcore_barrier()

    @pl.when(s == 0)
    def _():
        pltpu.sync_copy(shared, merged)
        pltpu.make_async_copy(user_hbm.at[uiv], urows, semu).wait()
        pltpu.make_async_copy(item_hbm.at[iiv], irows, semi).wait()

        lanes = lax.iota(jnp.int32, 16)
        svec = jnp.zeros((16,), jnp.float32)
        for i in range(8):
            tot = jnp.zeros((16,), jnp.float32)
            for g in range(NG):
                sl = pl.ds(16 * g, 16)
                pooled = jnp.maximum(merged[2 * i, sl], merged[2 * i + 1, sl])
                hyb = jnp.maximum(pooled, urows[i, sl])
                tot = tot + hyb * irows[i, sl]
            for sh in (8, 4, 2, 1):
                tot = tot + tot.at[lanes ^ sh].get(mode="promise_in_bounds")
            svec = jnp.where(lanes == i, tot, svec)
        outv[...] = svec
        pltpu.sync_copy(outv.at[pl.ds(0, 8)], out_hbm.at[pl.ds(c * 8, 8)])


@jax.jit
def kernel(user_table, item_table, user_inputs, L_inputs, S_inputs,
           item_inputs):
    batch = user_inputs.shape[0]
    idx_all = jnp.concatenate(
        [L_inputs.astype(jnp.int32)[:, None, :],
         S_inputs.astype(jnp.int32)[:, None, :]], axis=1)
    idx_all = idx_all.reshape(2 * batch, NCHUNK, CHUNK)

    mesh = plsc.VectorSubcoreMesh(core_axis_name="c", subcore_axis_name="s")
    out = pl.kernel(
        _hrm_sc,
        out_type=jax.ShapeDtypeStruct((batch,), jnp.float32),
        mesh=mesh,
        scratch_types=[
            pltpu.VMEM((NCHUNK, CHUNK), jnp.int32),
            pltpu.VMEM((CHUNK, EMB), jnp.float32),
            pltpu.VMEM((CHUNK, EMB), jnp.float32),
            pltpu.VMEM((CHUNK, EMB), jnp.float32),
            pltpu.VMEM((CHUNK, EMB), jnp.float32),
            pltpu.VMEM((EMB,), jnp.float32),
            pltpu.VMEM_SHARED((16, EMB), jnp.float32),
            pltpu.VMEM((16, EMB), jnp.float32),
            pltpu.VMEM((8,), jnp.int32),
            pltpu.VMEM((8,), jnp.int32),
            pltpu.VMEM((8, EMB), jnp.float32),
            pltpu.VMEM((8, EMB), jnp.float32),
            pltpu.VMEM((16,), jnp.float32),
            pltpu.SemaphoreType.DMA,
            pltpu.SemaphoreType.DMA,
            pltpu.SemaphoreType.DMA,
            pltpu.SemaphoreType.DMA,
            pltpu.SemaphoreType.DMA,
            pltpu.SemaphoreType.DMA,
        ],
    )(item_table, user_table, idx_all,
      user_inputs.astype(jnp.int32), item_inputs.astype(jnp.int32))
    return out.reshape(batch, 1)
